# baseline (speedup 1.0000x reference)
import functools

import jax
import jax.numpy as jnp
from jax import lax
from jax.experimental import pallas as pl
from jax.experimental.pallas import tpu as pltpu
from jax.experimental.pallas import tpu_sc as plsc

B = 16384
D = 64
H1 = 128
H2 = 64
NROWS = 1000000

NC = 2
NS = 16
NW = NC * NS

CW = 512
NCHUNK = NROWS // CW
TAILK = NCHUNK
TAILW = NROWS - NCHUNK * CW
MPW = 62
CAP = B + 16

_sc_mesh = plsc.VectorSubcoreMesh(core_axis_name="c", subcore_axis_name="s")


@functools.partial(
    pl.kernel,
    out_type=[
        jax.ShapeDtypeStruct((D * B,), jnp.float32),
        jax.ShapeDtypeStruct((D * B,), jnp.float32),
    ],
    mesh=_sc_mesh,
    compiler_params=pltpu.CompilerParams(
        disable_bounds_checks=True, needs_layout_passes=False
    ),
    scratch_types=[
        pltpu.VMEM((1024,), jnp.int32),
        pltpu.VMEM((CAP,), jnp.int32),
        pltpu.VMEM((CAP,), jnp.int32),
        pltpu.VMEM((CAP,), jnp.int32),
        pltpu.VMEM((D, CW), jnp.float32),
        pltpu.VMEM((D * 16,), jnp.float32),
        pltpu.VMEM((D * 16,), jnp.int32),
        pltpu.SemaphoreType.DMA,
    ],
)
def _sc_gather(uidx_hbm, iidx_hbm, utabT_hbm, itabT_hbm, uflat_hbm, iflat_hbm,
               idxbuf_v, rlist_v, blist_v, clist_v, blk_v, vals_v, offs_v,
               sem):
    wid = lax.axis_index("s") * NC + lax.axis_index("c")
    lanes = lax.iota(jnp.int32, 16)

    def run_table(idx_hbm, tab_hbm, out_hbm):
        def filt(g, cnt):
            pltpu.sync_copy(idx_hbm.at[pl.ds(g * 1024, 1024)], idxbuf_v)

            def grp(h, cnt):
                r = idxbuf_v[pl.ds(h * 16, 16)]
                bpos = g * 1024 + h * 16 + lanes
                mask = (lax.shift_right_logical(r, 9) & (NW - 1)) == wid
                plsc.store_compressed(rlist_v.at[pl.ds(cnt, 16)], r,
                                      mask=mask)
                plsc.store_compressed(blist_v.at[pl.ds(cnt, 16)], bpos,
                                      mask=mask)
                pc = plsc.all_reduce_population_count(mask)
                return cnt + pc[0]

            return lax.fori_loop(0, 64, grp, cnt, unroll=False)

        nlist = lax.fori_loop(0, B // 1024, filt, jnp.int32(0), unroll=False)
        nlist_g = (nlist + 15) // 16

        def serve_chunk(k, width):
            def cscan(h, cnt):
                r = rlist_v[pl.ds(h * 16, 16)]
                b = blist_v[pl.ds(h * 16, 16)]
                valid = (h * 16 + lanes) < nlist
                hit = valid & (lax.shift_right_logical(r, 9) == k)
                packed = b * CW + (r & (CW - 1))
                plsc.store_compressed(clist_v.at[pl.ds(cnt, 16)], packed,
                                      mask=hit)
                pc = plsc.all_reduce_population_count(hit)
                return cnt + pc[0]

            nc = lax.fori_loop(0, nlist_g, cscan, jnp.int32(0), unroll=False)

            @pl.when(nc > 0)
            def _():
                pltpu.sync_copy(
                    tab_hbm.at[
                        :, pl.ds(pl.multiple_of(k * CW, CW), width)
                    ],
                    blk_v.at[:, pl.ds(0, width)],
                )

                def sgrp(h, _):
                    packed = clist_v[pl.ds(h * 16, 16)]
                    live = (h * 16 + lanes) < nc
                    q = packed & (CW - 1)
                    b = lax.shift_right_logical(packed, 9)
                    for c in range(D):
                        vals = plsc.load_gather(
                            blk_v, [jnp.full((16,), c, jnp.int32), q]
                        )
                        vals_v[pl.ds(c * 16, 16)] = vals
                        offs_v[pl.ds(c * 16, 16)] = jnp.where(
                            live, b * D + c, -1
                        )
                    pltpu.async_copy(
                        vals_v,
                        out_hbm.at[plsc.Indices(offs_v, ignored_value=-1)],
                        sem,
                    ).wait()
                    return ()

                lax.fori_loop(0, (nc + 15) // 16, sgrp, (), unroll=False)

        def chunk_loop(m, _):
            k = wid + NW * m

            @pl.when(k < NCHUNK)
            def _():
                serve_chunk(k, CW)

            @pl.when(k == TAILK)
            def _():
                serve_chunk(k, 128)

            return ()

        lax.fori_loop(0, MPW, chunk_loop, (), unroll=False)

    run_table(uidx_hbm, utabT_hbm, uflat_hbm)
    run_table(iidx_hbm, itabT_hbm, iflat_hbm)


BBT = 2048


def _mlp_body(u_ref, i_ref, w1u_ref, w1i_ref, b1_ref, w2_ref, b2_ref,
              w3_ref, b3_ref, out_ref):
    h1 = u_ref[...] @ w1u_ref[...] + i_ref[...] @ w1i_ref[...] + b1_ref[...]
    h1 = jnp.maximum(h1, 0.0)
    h2 = jnp.maximum(h1 @ w2_ref[...] + b2_ref[...], 0.0)
    o = h2 @ w3_ref[...] + b3_ref[...]
    out_ref[...] = 1.0 / (1.0 + jnp.exp(-o))


def _mlp(uT, iT, w1u, w1i, b1, w2, b2, w3, b3):
    full = lambda i: (0, 0)
    return pl.pallas_call(
        _mlp_body,
        grid=(B // BBT,),
        in_specs=[
            pl.BlockSpec((BBT, D), lambda i: (i, 0)),
            pl.BlockSpec((BBT, D), lambda i: (i, 0)),
            pl.BlockSpec((D, H1), full),
            pl.BlockSpec((D, H1), full),
            pl.BlockSpec((1, H1), full),
            pl.BlockSpec((H1, H2), full),
            pl.BlockSpec((1, H2), full),
            pl.BlockSpec((H2, 1), full),
            pl.BlockSpec((1, 1), full),
        ],
        out_specs=pl.BlockSpec((BBT, 1), lambda i: (i, 0)),
        out_shape=jax.ShapeDtypeStruct((B, 1), jnp.float32),
    )(uT, iT, w1u, w1i, b1, w2, b2, w3, b3)


def kernel(user_input, item_input, user_table, item_table,
           W1, b1, W2, b2, W3, b3):
    utabT = user_table.T
    itabT = item_table.T
    uflat, iflat = _sc_gather(user_input, item_input, utabT, itabT)
    u = uflat.reshape(B, D)
    i = iflat.reshape(B, D)
    out = _mlp(u, i, W1[:, :D].T, W1[:, D:].T, b1.reshape(1, H1),
               W2.T, b2.reshape(1, H2), W3.T, b3.reshape(1, 1))
    return out


# speedup vs baseline: 6.8236x; 6.8236x over previous
import functools

import jax
import jax.numpy as jnp
from jax import lax
from jax.experimental import pallas as pl
from jax.experimental.pallas import tpu as pltpu
from jax.experimental.pallas import tpu_sc as plsc

B = 16384
D = 64
H1 = 128
H2 = 64
NROWS = 1000000

NC = 2
NS = 16
NW = NC * NS

CW = 512
NCHUNK = NROWS // CW
TAILK = NCHUNK
TAILW = NROWS - NCHUNK * CW
MPW = 62
CAP = B + 16
ZR = 256
SPR = B // NS

_sc_mesh = plsc.VectorSubcoreMesh(core_axis_name="c", subcore_axis_name="s")


@functools.partial(
    pl.kernel,
    out_type=[
        jax.ShapeDtypeStruct((NC, B, 128), jnp.float32),
        jax.ShapeDtypeStruct((NC, B, 128), jnp.float32),
    ],
    mesh=_sc_mesh,
    compiler_params=pltpu.CompilerParams(
        disable_bounds_checks=True, needs_layout_passes=False
    ),
    scratch_types=[
        pltpu.VMEM((1024,), jnp.int32),
        pltpu.VMEM((CAP,), jnp.int32),
        pltpu.VMEM((CAP,), jnp.int32),
        pltpu.VMEM((CAP,), jnp.int32),
        pltpu.VMEM((D, CW), jnp.float32),
        pltpu.VMEM((16, 128), jnp.float32),
        pltpu.VMEM((16,), jnp.int32),
        pltpu.VMEM((ZR, 128), jnp.float32),
        pltpu.SemaphoreType.DMA,
    ],
)
def _sc_gather(uidx_hbm, iidx_hbm, utabT_hbm, itabT_hbm, uout_hbm, iout_hbm,
               idxbuf_v, rlist_v, blist_v, clist_v, blk_v, vals_v, boffs_v,
               zbuf_v, sem):
    cid = lax.axis_index("c")
    sid = lax.axis_index("s")
    wid = sid * NC + cid
    lanes = lax.iota(jnp.int32, 16)

    zeros16 = jnp.zeros((16,), jnp.float32)

    def zb(i, _):
        zbuf_v[i // 8, pl.ds((i % 8) * 16, 16)] = zeros16
        return ()

    lax.fori_loop(0, ZR * 8, zb, (), unroll=False)

    def run_table(idx_hbm, tab_hbm, out_hbm):
        def zstripe(i, _):
            pltpu.sync_copy(
                zbuf_v,
                out_hbm.at[cid, pl.ds(sid * SPR + i * ZR, ZR), :],
            )
            return ()

        lax.fori_loop(0, SPR // ZR, zstripe, (), unroll=False)
        plsc.subcore_barrier()

        def filt(g, cnt):
            pltpu.sync_copy(idx_hbm.at[pl.ds(g * 1024, 1024)], idxbuf_v)

            def grp(h, cnt):
                r = idxbuf_v[pl.ds(h * 16, 16)]
                bpos = g * 1024 + h * 16 + lanes
                mask = (lax.shift_right_logical(r, 9) & (NW - 1)) == wid
                plsc.store_compressed(rlist_v.at[pl.ds(cnt, 16)], r,
                                      mask=mask)
                plsc.store_compressed(blist_v.at[pl.ds(cnt, 16)], bpos,
                                      mask=mask)
                pc = plsc.all_reduce_population_count(mask)
                return cnt + pc[0]

            return lax.fori_loop(0, 64, grp, cnt, unroll=False)

        nlist = lax.fori_loop(0, B // 1024, filt, jnp.int32(0), unroll=False)
        nlist_g = (nlist + 15) // 16

        def serve_chunk(k, width):
            def cscan(h, cnt):
                r = rlist_v[pl.ds(h * 16, 16)]
                b = blist_v[pl.ds(h * 16, 16)]
                valid = (h * 16 + lanes) < nlist
                hit = valid & (lax.shift_right_logical(r, 9) == k)
                packed = b * CW + (r & (CW - 1))
                plsc.store_compressed(clist_v.at[pl.ds(cnt, 16)], packed,
                                      mask=hit)
                pc = plsc.all_reduce_population_count(hit)
                return cnt + pc[0]

            nc = lax.fori_loop(0, nlist_g, cscan, jnp.int32(0), unroll=False)

            @pl.when(nc > 0)
            def _():
                pltpu.sync_copy(
                    tab_hbm.at[
                        :, pl.ds(pl.multiple_of(k * CW, CW), width)
                    ],
                    blk_v.at[:, pl.ds(0, width)],
                )

                def sgrp(h, _):
                    packed = clist_v[pl.ds(h * 16, 16)]
                    live = (h * 16 + lanes) < nc
                    q = packed & (CW - 1)
                    b = lax.shift_right_logical(packed, 9)
                    boffs_v[pl.ds(0, 16)] = jnp.where(live, b, -1)
                    for c in range(D):
                        vals = plsc.load_gather(
                            blk_v, [jnp.full((16,), c, jnp.int32), q]
                        )
                        plsc.store_scatter(
                            vals_v, [lanes, jnp.full((16,), c, jnp.int32)],
                            vals,
                        )
                    pltpu.async_copy(
                        vals_v,
                        out_hbm.at[cid].at[
                            plsc.Indices(boffs_v, ignored_value=-1)
                        ],
                        sem,
                    ).wait()
                    return ()

                lax.fori_loop(0, (nc + 15) // 16, sgrp, (), unroll=False)

        def chunk_loop(m, _):
            k = wid + NW * m

            @pl.when(k < NCHUNK)
            def _():
                serve_chunk(k, CW)

            @pl.when(k == TAILK)
            def _():
                serve_chunk(k, 128)

            return ()

        lax.fori_loop(0, MPW, chunk_loop, (), unroll=False)

    run_table(uidx_hbm, utabT_hbm, uout_hbm)
    run_table(iidx_hbm, itabT_hbm, iout_hbm)


BBT = 2048


def _mlp_body(u_ref, i_ref, w1u_ref, w1i_ref, b1_ref, w2_ref, b2_ref,
              w3_ref, b3_ref, out_ref):
    u = u_ref[0, :, :D] + u_ref[1, :, :D]
    i = i_ref[0, :, :D] + i_ref[1, :, :D]

    h1 = u @ w1u_ref[...] + i @ w1i_ref[...] + b1_ref[...]
    h1 = jnp.maximum(h1, 0.0)
    h2 = jnp.maximum(h1 @ w2_ref[...] + b2_ref[...], 0.0)
    o = h2 @ w3_ref[...] + b3_ref[...]
    out_ref[...] = 1.0 / (1.0 + jnp.exp(-o))


def _mlp(u2, i2, w1u, w1i, b1, w2, b2, w3, b3):
    full = lambda i: (0, 0)
    full3 = lambda i: (0, i, 0)
    return pl.pallas_call(
        _mlp_body,
        grid=(B // BBT,),
        in_specs=[
            pl.BlockSpec((NC, BBT, 128), full3),
            pl.BlockSpec((NC, BBT, 128), full3),
            pl.BlockSpec((D, H1), full),
            pl.BlockSpec((D, H1), full),
            pl.BlockSpec((1, H1), full),
            pl.BlockSpec((H1, H2), full),
            pl.BlockSpec((1, H2), full),
            pl.BlockSpec((H2, 1), full),
            pl.BlockSpec((1, 1), full),
        ],
        out_specs=pl.BlockSpec((BBT, 1), lambda i: (i, 0)),
        out_shape=jax.ShapeDtypeStruct((B, 1), jnp.float32),
    )(u2, i2, w1u, w1i, b1, w2, b2, w3, b3)


def kernel(user_input, item_input, user_table, item_table,
           W1, b1, W2, b2, W3, b3):
    utabT = user_table.T
    itabT = item_table.T
    u2, i2 = _sc_gather(user_input, item_input, utabT, itabT)
    return _mlp(u2, i2, W1[:, :D].T, W1[:, D:].T, b1.reshape(1, H1),
                W2.T, b2.reshape(1, H2), W3.T, b3.reshape(1, 1))


# speedup vs baseline: 7.0652x; 1.0354x over previous
import functools

import jax
import jax.numpy as jnp
from jax import lax
from jax.experimental import pallas as pl
from jax.experimental.pallas import tpu as pltpu
from jax.experimental.pallas import tpu_sc as plsc

B = 16384
D = 64
H1 = 128
H2 = 64
NROWS = 1000000

NC = 2
NS = 16
NW = NC * NS

CW = 512
NCHUNK = NROWS // CW
TAILK = NCHUNK
TAILW = NROWS - NCHUNK * CW
MPW = 62
CAP = B + 16
ZR = 256
SPR = B // NS

_sc_mesh = plsc.VectorSubcoreMesh(core_axis_name="c", subcore_axis_name="s")


@functools.partial(
    pl.kernel,
    out_type=[
        jax.ShapeDtypeStruct((NC, B, 128), jnp.float32),
        jax.ShapeDtypeStruct((NC, B, 128), jnp.float32),
    ],
    mesh=_sc_mesh,
    compiler_params=pltpu.CompilerParams(
        disable_bounds_checks=True, needs_layout_passes=False
    ),
    scratch_types=[
        pltpu.VMEM((1024,), jnp.int32),
        pltpu.VMEM((CAP,), jnp.int32),
        pltpu.VMEM((CAP,), jnp.int32),
        pltpu.VMEM((CAP,), jnp.int32),
        pltpu.VMEM((D, CW), jnp.float32),
        pltpu.VMEM((16, 128), jnp.float32),
        pltpu.VMEM((16, 128), jnp.float32),
        pltpu.VMEM((16, 128), jnp.float32),
        pltpu.VMEM((16, 128), jnp.float32),
        pltpu.VMEM((16,), jnp.int32),
        pltpu.VMEM((16,), jnp.int32),
        pltpu.VMEM((16,), jnp.int32),
        pltpu.VMEM((16,), jnp.int32),
        pltpu.VMEM((ZR, 128), jnp.float32),
        pltpu.SemaphoreType.DMA,
        pltpu.SemaphoreType.DMA,
        pltpu.SemaphoreType.DMA,
        pltpu.SemaphoreType.DMA,
        pltpu.SemaphoreType.DMA,
    ],
)
def _sc_gather(uidx_hbm, iidx_hbm, utabT_hbm, itabT_hbm, uout_hbm, iout_hbm,
               idxbuf_v, rlist_v, blist_v, clist_v, blk_v,
               vals0_v, vals1_v, vals2_v, vals3_v,
               boffs0_v, boffs1_v, boffs2_v, boffs3_v,
               zbuf_v, sem, ssem0, ssem1, ssem2, ssem3):
    cid = lax.axis_index("c")
    sid = lax.axis_index("s")
    wid = sid * NC + cid
    lanes = lax.iota(jnp.int32, 16)
    vals_js = (vals0_v, vals1_v, vals2_v, vals3_v)
    boffs_js = (boffs0_v, boffs1_v, boffs2_v, boffs3_v)
    ssems = (ssem0, ssem1, ssem2, ssem3)

    zeros16 = jnp.zeros((16,), jnp.float32)

    def zb(i, _):
        zbuf_v[i // 8, pl.ds((i % 8) * 16, 16)] = zeros16
        return ()

    lax.fori_loop(0, ZR * 8, zb, (), unroll=False)

    def run_table(idx_hbm, tab_hbm, out_hbm, ucnts):
        def zstripe(i, _):
            pltpu.sync_copy(
                zbuf_v,
                out_hbm.at[cid, pl.ds(sid * SPR + i * ZR, ZR), :],
            )
            return ()

        lax.fori_loop(0, SPR // ZR, zstripe, (), unroll=False)
        plsc.subcore_barrier()

        def filt(g, cnt):
            pltpu.sync_copy(idx_hbm.at[pl.ds(g * 1024, 1024)], idxbuf_v)

            def grp(h, cnt):
                r = idxbuf_v[pl.ds(h * 16, 16)]
                bpos = g * 1024 + h * 16 + lanes
                mask = (lax.shift_right_logical(r, 9) & (NW - 1)) == wid
                plsc.store_compressed(rlist_v.at[pl.ds(cnt, 16)], r,
                                      mask=mask)
                plsc.store_compressed(blist_v.at[pl.ds(cnt, 16)], bpos,
                                      mask=mask)
                pc = plsc.all_reduce_population_count(mask)
                return cnt + pc[0]

            return lax.fori_loop(0, 64, grp, cnt, unroll=False)

        nlist = lax.fori_loop(0, B // 1024, filt, jnp.int32(0), unroll=False)
        nlist_g = (nlist + 15) // 16

        def chunk_loop(m, ucnts):
            k = wid + NW * m

            def cscan(h, cnt):
                r = rlist_v[pl.ds(h * 16, 16)]
                b = blist_v[pl.ds(h * 16, 16)]
                valid = (h * 16 + lanes) < nlist
                hit = valid & (lax.shift_right_logical(r, 9) == k)
                packed = b * CW + (r & (CW - 1))
                plsc.store_compressed(clist_v.at[pl.ds(cnt, 16)], packed,
                                      mask=hit)
                pc = plsc.all_reduce_population_count(hit)
                return cnt + pc[0]

            nc = lax.fori_loop(0, nlist_g, cscan, jnp.int32(0), unroll=False)

            @pl.when((nc > 0) & (k < NCHUNK))
            def _():
                pltpu.sync_copy(
                    tab_hbm.at[:, pl.ds(pl.multiple_of(k * CW, CW), CW)],
                    blk_v,
                )

            @pl.when((nc > 0) & (k == TAILK))
            def _():
                pltpu.sync_copy(
                    tab_hbm.at[:, pl.ds(pl.multiple_of(k * CW, CW), 128)],
                    blk_v.at[:, pl.ds(0, 128)],
                )

            def sgrp4(o, ucnts):
                newu = []
                for j in range(4):
                    h = 4 * o + j
                    grp_live = h * 16 < nc
                    uj = ucnts[j]

                    @pl.when(grp_live)
                    def _(j=j, h=h, uj=uj):
                        @pl.when(uj > 0)
                        def _():
                            pltpu.make_async_copy(
                                vals_js[j],
                                out_hbm.at[cid].at[
                                    plsc.Indices(boffs_js[j],
                                                 ignored_value=-1)
                                ],
                                ssems[j],
                            ).wait()

                        packed = clist_v[pl.ds(h * 16, 16)]
                        live = (h * 16 + lanes) < nc
                        q = packed & (CW - 1)
                        b = lax.shift_right_logical(packed, 9)
                        boffs_js[j][pl.ds(0, 16)] = jnp.where(live, b, -1)
                        for c in range(D):
                            vals = plsc.load_gather(
                                blk_v, [jnp.full((16,), c, jnp.int32), q]
                            )
                            plsc.store_scatter(
                                vals_js[j],
                                [lanes, jnp.full((16,), c, jnp.int32)],
                                vals,
                            )
                        pltpu.async_copy(
                            vals_js[j],
                            out_hbm.at[cid].at[
                                plsc.Indices(boffs_js[j], ignored_value=-1)
                            ],
                            ssems[j],
                        )

                    newu.append(uj + grp_live.astype(jnp.int32))
                return tuple(newu)

            return lax.fori_loop(0, (nc + 63) // 64, sgrp4, ucnts,
                                 unroll=False)

        return lax.fori_loop(0, MPW, chunk_loop, ucnts, unroll=False)

    ucnts = (jnp.int32(0),) * 4
    ucnts = run_table(uidx_hbm, utabT_hbm, uout_hbm, ucnts)
    ucnts = run_table(iidx_hbm, itabT_hbm, iout_hbm, ucnts)

    for j in range(4):
        @pl.when(ucnts[j] > 0)
        def _(j=j):
            pltpu.make_async_copy(
                vals_js[j],
                iout_hbm.at[cid].at[
                    plsc.Indices(boffs_js[j], ignored_value=-1)
                ],
                ssems[j],
            ).wait()


BBT = 2048


def _mlp_body(u_ref, i_ref, w1u_ref, w1i_ref, b1_ref, w2_ref, b2_ref,
              w3_ref, b3_ref, out_ref):
    u = u_ref[0, :, :D] + u_ref[1, :, :D]
    i = i_ref[0, :, :D] + i_ref[1, :, :D]

    h1 = u @ w1u_ref[...] + i @ w1i_ref[...] + b1_ref[...]
    h1 = jnp.maximum(h1, 0.0)
    h2 = jnp.maximum(h1 @ w2_ref[...] + b2_ref[...], 0.0)
    o = h2 @ w3_ref[...] + b3_ref[...]
    out_ref[...] = 1.0 / (1.0 + jnp.exp(-o))


def _mlp(u2, i2, w1u, w1i, b1, w2, b2, w3, b3):
    full = lambda i: (0, 0)
    full3 = lambda i: (0, i, 0)
    return pl.pallas_call(
        _mlp_body,
        grid=(B // BBT,),
        in_specs=[
            pl.BlockSpec((NC, BBT, 128), full3),
            pl.BlockSpec((NC, BBT, 128), full3),
            pl.BlockSpec((D, H1), full),
            pl.BlockSpec((D, H1), full),
            pl.BlockSpec((1, H1), full),
            pl.BlockSpec((H1, H2), full),
            pl.BlockSpec((1, H2), full),
            pl.BlockSpec((H2, 1), full),
            pl.BlockSpec((1, 1), full),
        ],
        out_specs=pl.BlockSpec((BBT, 1), lambda i: (i, 0)),
        out_shape=jax.ShapeDtypeStruct((B, 1), jnp.float32),
    )(u2, i2, w1u, w1i, b1, w2, b2, w3, b3)


def kernel(user_input, item_input, user_table, item_table,
           W1, b1, W2, b2, W3, b3):
    utabT = user_table.T
    itabT = item_table.T
    u2, i2 = _sc_gather(user_input, item_input, utabT, itabT)
    return _mlp(u2, i2, W1[:, :D].T, W1[:, D:].T, b1.reshape(1, H1),
                W2.T, b2.reshape(1, H2), W3.T, b3.reshape(1, 1))
